# baseline (speedup 1.0000x reference)
import jax
import jax.numpy as jnp
from jax import lax
from jax.experimental import pallas as pl
from jax.experimental.pallas import tpu as pltpu
from jax.experimental.pallas import tpu_sc as plsc

N = 10000
D = 128
G = 64
ING = 128
H = 256
OUT = 128

NC = 2
NS = 16
SLICE = 128
SLICES_PER_W = 3
CHUNK = SLICE * SLICES_PER_W
N_MAIN = (N // CHUNK) * CHUNK
ACTIVE = N_MAIN // CHUNK
N_TAIL = N - N_MAIN


def _sc_seg_sum(x_hbm, batch_hbm, zeros_hbm, out_hbm, idx_v, rows_v, shared):
    c = lax.axis_index("c")
    s = lax.axis_index("s")
    w = s * NC + c

    @pl.when(s == 0)
    def _zero():
        pltpu.sync_copy(zeros_hbm, shared)

    plsc.subcore_barrier()

    @pl.when(w < ACTIVE)
    def _accumulate():
        pltpu.sync_copy(x_hbm.at[pl.ds(w * CHUNK, CHUNK)], rows_v)
        pltpu.sync_copy(batch_hbm.at[w], idx_v)
        for j in range(SLICES_PER_W):
            pltpu.sync_copy(rows_v.at[pl.ds(j * SLICE, SLICE)],
                            shared.at[idx_v.at[j]], add=True)

    plsc.subcore_barrier()

    @pl.when(s == 0)
    def _publish():
        pltpu.sync_copy(shared, out_hbm.at[c])


def _tc_finish(part_ref, batch_ref, xtail_ref, u_ref, w1_ref, b1_ref,
               w2_ref, b2_ref, w3_ref, b3_ref, out_ref):
    batch = batch_ref[0, :]
    seg_ids = jax.lax.broadcasted_iota(jnp.int32, (G, N), 0)
    onehot = (batch[None, :] == seg_ids).astype(jnp.float32)
    cnt = jnp.sum(onehot, axis=1, keepdims=True)
    seg_sum = part_ref[0] + part_ref[1]
    seg_sum = seg_sum + jnp.dot(onehot[:, N_MAIN:], xtail_ref[...],
                                preferred_element_type=jnp.float32)
    seg_mean = seg_sum / jnp.maximum(cnt, 1.0)
    cat = jnp.concatenate([u_ref[...], seg_mean], axis=1)
    h = jnp.dot(cat, w1_ref[...], preferred_element_type=jnp.float32)
    h = jnp.maximum(h + b1_ref[...], 0.0)
    h = jnp.dot(h, w2_ref[...], preferred_element_type=jnp.float32)
    h = jnp.maximum(h + b2_ref[...], 0.0)
    h = jnp.dot(h, w3_ref[...], preferred_element_type=jnp.float32)
    out_ref[...] = h + b3_ref[...]


_sc_call = pl.kernel(
    _sc_seg_sum,
    out_type=jax.ShapeDtypeStruct((NC, G, D), jnp.float32),
    mesh=plsc.VectorSubcoreMesh(core_axis_name="c", subcore_axis_name="s",
                                num_cores=NC, num_subcores=NS),
    scratch_types=[
        pltpu.VMEM((SLICES_PER_W, SLICE), jnp.int32),
        pltpu.VMEM((CHUNK, D), jnp.float32),
        pltpu.VMEM_SHARED((G, D), jnp.float32),
    ],
)


def kernel(x, edge_index, u, batch, W1, b1, W2, b2, W3, b3):
    del edge_index
    batch_r = batch[:N_MAIN].reshape(ACTIVE, SLICES_PER_W, SLICE)
    zeros = jnp.zeros((G, D), jnp.float32)
    partials = _sc_call(x, batch_r, zeros)
    tc_args = (partials, batch.reshape(1, N), x[N_MAIN:], u, W1.T,
               b1.reshape(1, H), W2.T, b2.reshape(1, H), W3.T,
               b3.reshape(1, OUT))
    return pl.pallas_call(
        _tc_finish,
        out_shape=jax.ShapeDtypeStruct((G, OUT), jnp.float32),
    )(*tc_args)


# speedup vs baseline: 1.0190x; 1.0190x over previous
import jax
import jax.numpy as jnp
from jax import lax
from jax.experimental import pallas as pl
from jax.experimental.pallas import tpu as pltpu
from jax.experimental.pallas import tpu_sc as plsc

N = 10000
D = 128
G = 64
ING = 128
H = 256
OUT = 128

NC = 2
NS = 16
SLICE = 128
SLICES_PER_W = 3
CHUNK = SLICE * SLICES_PER_W
N_MAIN = (N // CHUNK) * CHUNK
ACTIVE = N_MAIN // CHUNK
N_TAIL = N - N_MAIN


def _sc_seg_sum(x_hbm, batch_hbm, out_hbm, idx_v, rows_v, zero_v, shared,
                sem0, sem1, sem2):
    c = lax.axis_index("c")
    s = lax.axis_index("s")
    w = s * NC + c
    sems = (sem0, sem1, sem2)

    @pl.when(s < G // 8)
    def _zero():
        for r in range(8):
            for k in range(D // 16):
                zero_v[r, pl.ds(k * 16, 16)] = jnp.zeros((16,), jnp.float32)
        pltpu.sync_copy(zero_v, shared.at[pl.ds(pl.multiple_of(s * 8, 8), 8)])

    plsc.subcore_barrier()

    @pl.when(w < ACTIVE)
    def _accumulate():
        pltpu.sync_copy(batch_hbm.at[w], idx_v)
        chunk = x_hbm.at[pl.ds(w * CHUNK, CHUNK)]
        loads = [pltpu.async_copy(chunk.at[pl.ds(j * SLICE, SLICE)],
                                  rows_v.at[pl.ds(j * SLICE, SLICE)], sems[j])
                 for j in range(SLICES_PER_W)]
        for j in range(SLICES_PER_W):
            loads[j].wait()
            pltpu.sync_copy(rows_v.at[pl.ds(j * SLICE, SLICE)],
                            shared.at[idx_v.at[j]], add=True)

    plsc.subcore_barrier()

    @pl.when(s == 0)
    def _publish():
        pltpu.sync_copy(shared, out_hbm.at[c])


def _tc_finish(part_ref, batch_ref, xtail_ref, u_ref, w1_ref, b1_ref,
               w2_ref, b2_ref, w3_ref, b3_ref, out_ref):
    batch = batch_ref[0, :]
    seg_ids = jax.lax.broadcasted_iota(jnp.int32, (G, N), 0)
    onehot = (batch[None, :] == seg_ids).astype(jnp.float32)
    cnt = jnp.sum(onehot, axis=1, keepdims=True)
    seg_sum = part_ref[0] + part_ref[1]
    seg_sum = seg_sum + jnp.dot(onehot[:, N_MAIN:], xtail_ref[...],
                                preferred_element_type=jnp.float32)
    seg_mean = seg_sum / jnp.maximum(cnt, 1.0)
    cat = jnp.concatenate([u_ref[...], seg_mean], axis=1)
    h = jnp.dot(cat, w1_ref[...], preferred_element_type=jnp.float32)
    h = jnp.maximum(h + b1_ref[...], 0.0)
    h = jnp.dot(h, w2_ref[...], preferred_element_type=jnp.float32)
    h = jnp.maximum(h + b2_ref[...], 0.0)
    h = jnp.dot(h, w3_ref[...], preferred_element_type=jnp.float32)
    out_ref[...] = h + b3_ref[...]


_sc_call = pl.kernel(
    _sc_seg_sum,
    out_type=jax.ShapeDtypeStruct((NC, G, D), jnp.float32),
    mesh=plsc.VectorSubcoreMesh(core_axis_name="c", subcore_axis_name="s",
                                num_cores=NC, num_subcores=NS),
    scratch_types=[
        pltpu.VMEM((SLICES_PER_W, SLICE), jnp.int32),
        pltpu.VMEM((CHUNK, D), jnp.float32),
        pltpu.VMEM((8, D), jnp.float32),
        pltpu.VMEM_SHARED((G, D), jnp.float32),
        pltpu.SemaphoreType.DMA,
        pltpu.SemaphoreType.DMA,
        pltpu.SemaphoreType.DMA,
    ],
)


def kernel(x, edge_index, u, batch, W1, b1, W2, b2, W3, b3):
    del edge_index
    batch_r = batch[:N_MAIN].reshape(ACTIVE, SLICES_PER_W, SLICE)
    partials = _sc_call(x, batch_r)
    tc_args = (partials, batch.reshape(1, N), x[N_MAIN:], u, W1.T,
               b1.reshape(1, H), W2.T, b2.reshape(1, H), W3.T,
               b3.reshape(1, OUT))
    return pl.pallas_call(
        _tc_finish,
        out_shape=jax.ShapeDtypeStruct((G, OUT), jnp.float32),
    )(*tc_args)


# speedup vs baseline: 1.1066x; 1.0859x over previous
import jax
import jax.numpy as jnp
from jax import lax
from jax.experimental import pallas as pl
from jax.experimental.pallas import tpu as pltpu
from jax.experimental.pallas import tpu_sc as plsc

N = 10000
D = 128
G = 64
ING = 128
H = 256
OUT = 128

NC = 2
NS = 16
NW = NC * NS
SLICE = 128
SLICES_PER_W = 2
HEAD_SLICES = 14
HEAD = HEAD_SLICES * SLICE
N_SC = NW * SLICES_PER_W * SLICE
N_MAIN = HEAD + N_SC
N_TAIL = N - N_MAIN


def _sc_seg_sum(x_hbm, batch_hbm, out_hbm, idx0, idx1, rows_v, zero_v,
                shared, sem0, sem1):
    c = lax.axis_index("c")
    s = lax.axis_index("s")
    w = s * NC + c

    @pl.when(s < G // 8)
    def _zero():
        for r in range(8):
            for k in range(D // 16):
                zero_v[r, pl.ds(k * 16, 16)] = jnp.zeros((16,), jnp.float32)
        pltpu.sync_copy(zero_v, shared.at[pl.ds(pl.multiple_of(s * 8, 8), 8)])

    plsc.subcore_barrier()

    t = HEAD_SLICES + w * SLICES_PER_W
    ld0 = pltpu.async_copy(x_hbm.at[pl.ds(t * SLICE, SLICE)],
                           rows_v.at[pl.ds(0, SLICE)], sem0)
    ld1 = pltpu.async_copy(x_hbm.at[pl.ds((t + 1) * SLICE, SLICE)],
                           rows_v.at[pl.ds(SLICE, SLICE)], sem1)
    pltpu.sync_copy(batch_hbm.at[pl.ds(t * SLICE, SLICE)], idx0)
    pltpu.sync_copy(batch_hbm.at[pl.ds((t + 1) * SLICE, SLICE)], idx1)
    ld0.wait()
    pltpu.sync_copy(rows_v.at[pl.ds(0, SLICE)], shared.at[idx0], add=True)
    ld1.wait()
    pltpu.sync_copy(rows_v.at[pl.ds(SLICE, SLICE)], shared.at[idx1], add=True)

    plsc.subcore_barrier()

    @pl.when(s == 0)
    def _publish():
        pltpu.sync_copy(shared, out_hbm.at[c])


_sc_call = pl.kernel(
    _sc_seg_sum,
    out_type=jax.ShapeDtypeStruct((NC, G, D), jnp.float32),
    mesh=plsc.VectorSubcoreMesh(core_axis_name="c", subcore_axis_name="s",
                                num_cores=NC, num_subcores=NS),
    scratch_types=[
        pltpu.VMEM((SLICE,), jnp.int32),
        pltpu.VMEM((SLICE,), jnp.int32),
        pltpu.VMEM((SLICES_PER_W * SLICE, D), jnp.float32),
        pltpu.VMEM((8, D), jnp.float32),
        pltpu.VMEM_SHARED((G, D), jnp.float32),
        pltpu.SemaphoreType.DMA,
        pltpu.SemaphoreType.DMA,
    ],
)


def _tc_partial(xh_ref, xtail_ref, batch_ref, part_ref, cnt_ref):
    batch = batch_ref[0, :]
    seg_ids = jax.lax.broadcasted_iota(jnp.int32, (G, N), 0)
    onehot = (batch[None, :] == seg_ids).astype(jnp.float32)
    cnt_ref[...] = jnp.sum(onehot, axis=1, keepdims=True) * jnp.ones(
        (1, D), jnp.float32)
    part = jnp.dot(onehot[:, :HEAD], xh_ref[...],
                   preferred_element_type=jnp.float32)
    part_ref[...] = part + jnp.dot(onehot[:, N_MAIN:], xtail_ref[...],
                                   preferred_element_type=jnp.float32)


def _tc_finish(sc_ref, tcp_ref, cnt_ref, u_ref, w1_ref, b1_ref,
               w2_ref, b2_ref, w3_ref, b3_ref, out_ref):
    seg_sum = sc_ref[0] + sc_ref[1] + tcp_ref[...]
    seg_mean = seg_sum / jnp.maximum(cnt_ref[...], 1.0)
    cat = jnp.concatenate([u_ref[...], seg_mean], axis=1)
    h = jnp.dot(cat, w1_ref[...], preferred_element_type=jnp.float32)
    h = jnp.maximum(h + b1_ref[...], 0.0)
    h = jnp.dot(h, w2_ref[...], preferred_element_type=jnp.float32)
    h = jnp.maximum(h + b2_ref[...], 0.0)
    h = jnp.dot(h, w3_ref[...], preferred_element_type=jnp.float32)
    out_ref[...] = h + b3_ref[...]


def kernel(x, edge_index, u, batch, W1, b1, W2, b2, W3, b3):
    del edge_index
    sc_part = _sc_call(x, batch)
    tc_part, cnt = pl.pallas_call(
        _tc_partial,
        grid=(1,),
        in_specs=[
            pl.BlockSpec((HEAD, D), lambda i: (0, 0)),
            pl.BlockSpec((N_TAIL, D), lambda i: (0, 0)),
            pl.BlockSpec((1, N), lambda i: (0, 0)),
        ],
        out_specs=[
            pl.BlockSpec((G, D), lambda i: (0, 0)),
            pl.BlockSpec((G, D), lambda i: (0, 0)),
        ],
        out_shape=[
            jax.ShapeDtypeStruct((G, D), jnp.float32),
            jax.ShapeDtypeStruct((G, D), jnp.float32),
        ],
    )(x, x[N_MAIN:], batch.reshape(1, N))
    args = (sc_part, tc_part, cnt, u, W1.T, b1.reshape(1, H), W2.T,
            b2.reshape(1, H), W3.T, b3.reshape(1, OUT))
    return pl.pallas_call(
        _tc_finish,
        out_shape=jax.ShapeDtypeStruct((G, OUT), jnp.float32),
    )(*args)


# speedup vs baseline: 1.1762x; 1.0629x over previous
import jax
import jax.numpy as jnp
from jax import lax
from jax.experimental import pallas as pl
from jax.experimental.pallas import tpu as pltpu
from jax.experimental.pallas import tpu_sc as plsc

N = 10000
D = 128
G = 64
ING = 128
H = 256
OUT = 128

NC = 1
NS = 16
NW = NC * NS
SLICE = 128
SLICES_PER_W = 2
HEAD_SLICES = 46
HEAD = HEAD_SLICES * SLICE
N_SC = NW * SLICES_PER_W * SLICE
N_MAIN = HEAD + N_SC
N_TAIL = N - N_MAIN


def _sc_seg_sum(x_hbm, batch_hbm, out_hbm, idx0, idx1, rows_v, zero_v,
                shared, sem0, sem1):
    c = lax.axis_index("c")
    s = lax.axis_index("s")
    w = s * NC + c

    @pl.when(s < G // 8)
    def _zero():
        for r in range(8):
            for k in range(D // 16):
                zero_v[r, pl.ds(k * 16, 16)] = jnp.zeros((16,), jnp.float32)
        pltpu.sync_copy(zero_v, shared.at[pl.ds(pl.multiple_of(s * 8, 8), 8)])

    plsc.subcore_barrier()

    t = HEAD_SLICES + w * SLICES_PER_W
    ld0 = pltpu.async_copy(x_hbm.at[pl.ds(t * SLICE, SLICE)],
                           rows_v.at[pl.ds(0, SLICE)], sem0)
    ld1 = pltpu.async_copy(x_hbm.at[pl.ds((t + 1) * SLICE, SLICE)],
                           rows_v.at[pl.ds(SLICE, SLICE)], sem1)
    pltpu.sync_copy(batch_hbm.at[pl.ds(t * SLICE, SLICE)], idx0)
    pltpu.sync_copy(batch_hbm.at[pl.ds((t + 1) * SLICE, SLICE)], idx1)
    ld0.wait()
    pltpu.sync_copy(rows_v.at[pl.ds(0, SLICE)], shared.at[idx0], add=True)
    ld1.wait()
    pltpu.sync_copy(rows_v.at[pl.ds(SLICE, SLICE)], shared.at[idx1], add=True)

    plsc.subcore_barrier()

    @pl.when(s == 0)
    def _publish():
        pltpu.sync_copy(shared, out_hbm.at[c])


_sc_call = pl.kernel(
    _sc_seg_sum,
    out_type=jax.ShapeDtypeStruct((NC, G, D), jnp.float32),
    mesh=plsc.VectorSubcoreMesh(core_axis_name="c", subcore_axis_name="s",
                                num_cores=NC, num_subcores=NS),
    scratch_types=[
        pltpu.VMEM((SLICE,), jnp.int32),
        pltpu.VMEM((SLICE,), jnp.int32),
        pltpu.VMEM((SLICES_PER_W * SLICE, D), jnp.float32),
        pltpu.VMEM((8, D), jnp.float32),
        pltpu.VMEM_SHARED((G, D), jnp.float32),
        pltpu.SemaphoreType.DMA,
        pltpu.SemaphoreType.DMA,
    ],
)


def _tc_partial(xh_ref, xtail_ref, batch_ref, part_ref, cnt_ref):
    batch = batch_ref[0, :]
    seg_ids = jax.lax.broadcasted_iota(jnp.int32, (G, N), 0)
    onehot = (batch[None, :] == seg_ids).astype(jnp.float32)
    cnt_ref[...] = jnp.sum(onehot, axis=1, keepdims=True) * jnp.ones(
        (1, D), jnp.float32)
    part = jnp.dot(onehot[:, :HEAD], xh_ref[...],
                   preferred_element_type=jnp.float32)
    part_ref[...] = part + jnp.dot(onehot[:, N_MAIN:], xtail_ref[...],
                                   preferred_element_type=jnp.float32)


def _tc_finish(sc_ref, tcp_ref, cnt_ref, u_ref, w1_ref, b1_ref,
               w2_ref, b2_ref, w3_ref, b3_ref, out_ref):
    seg_sum = jnp.sum(sc_ref[...], axis=0) + tcp_ref[...]
    seg_mean = seg_sum / jnp.maximum(cnt_ref[...], 1.0)
    cat = jnp.concatenate([u_ref[...], seg_mean], axis=1)
    h = jnp.dot(cat, w1_ref[...], preferred_element_type=jnp.float32)
    h = jnp.maximum(h + b1_ref[...], 0.0)
    h = jnp.dot(h, w2_ref[...], preferred_element_type=jnp.float32)
    h = jnp.maximum(h + b2_ref[...], 0.0)
    h = jnp.dot(h, w3_ref[...], preferred_element_type=jnp.float32)
    out_ref[...] = h + b3_ref[...]


def kernel(x, edge_index, u, batch, W1, b1, W2, b2, W3, b3):
    del edge_index
    sc_part = _sc_call(x, batch)
    tc_part, cnt = pl.pallas_call(
        _tc_partial,
        grid=(1,),
        in_specs=[
            pl.BlockSpec((HEAD, D), lambda i: (0, 0)),
            pl.BlockSpec((N_TAIL, D), lambda i: (0, 0)),
            pl.BlockSpec((1, N), lambda i: (0, 0)),
        ],
        out_specs=[
            pl.BlockSpec((G, D), lambda i: (0, 0)),
            pl.BlockSpec((G, D), lambda i: (0, 0)),
        ],
        out_shape=[
            jax.ShapeDtypeStruct((G, D), jnp.float32),
            jax.ShapeDtypeStruct((G, D), jnp.float32),
        ],
    )(x, x[N_MAIN:], batch.reshape(1, N))
    args = (sc_part, tc_part, cnt, u, W1.T, b1.reshape(1, H), W2.T,
            b2.reshape(1, H), W3.T, b3.reshape(1, OUT))
    return pl.pallas_call(
        _tc_finish,
        out_shape=jax.ShapeDtypeStruct((G, OUT), jnp.float32),
    )(*args)


# speedup vs baseline: 1.1775x; 1.0011x over previous
import jax
import jax.numpy as jnp
from jax import lax
from jax.experimental import pallas as pl
from jax.experimental.pallas import tpu as pltpu
from jax.experimental.pallas import tpu_sc as plsc

N = 10000
D = 128
G = 64
ING = 128
H = 256
OUT = 128

NC = 1
NS = 16
NW = NC * NS
SLICE = 128
SLICES_PER_W = 2
HEAD_SLICES = 46
HEAD = HEAD_SLICES * SLICE
N_SC = NW * SLICES_PER_W * SLICE
N_MAIN = HEAD + N_SC
N_TAIL = N - N_MAIN


def _sc_seg_sum(x_hbm, batch_hbm, out_hbm, idx0, idx1, rows_v, zero_v,
                shared, sem0, sem1):
    c = lax.axis_index("c")
    s = lax.axis_index("s")
    w = s * NC + c

    t = HEAD_SLICES + w * SLICES_PER_W
    ld0 = pltpu.async_copy(x_hbm.at[pl.ds(t * SLICE, SLICE)],
                           rows_v.at[pl.ds(0, SLICE)], sem0)
    ld1 = pltpu.async_copy(x_hbm.at[pl.ds((t + 1) * SLICE, SLICE)],
                           rows_v.at[pl.ds(SLICE, SLICE)], sem1)
    pltpu.sync_copy(batch_hbm.at[pl.ds(t * SLICE, SLICE)], idx0)
    pltpu.sync_copy(batch_hbm.at[pl.ds((t + 1) * SLICE, SLICE)], idx1)

    @pl.when(s < G // 8)
    def _zero():
        for r in range(8):
            for k in range(D // 16):
                zero_v[r, pl.ds(k * 16, 16)] = jnp.zeros((16,), jnp.float32)
        pltpu.sync_copy(zero_v, shared.at[pl.ds(pl.multiple_of(s * 8, 8), 8)])

    plsc.subcore_barrier()

    ld0.wait()
    pltpu.sync_copy(rows_v.at[pl.ds(0, SLICE)], shared.at[idx0], add=True)
    ld1.wait()
    pltpu.sync_copy(rows_v.at[pl.ds(SLICE, SLICE)], shared.at[idx1], add=True)

    plsc.subcore_barrier()

    @pl.when(s == 0)
    def _publish():
        pltpu.sync_copy(shared, out_hbm.at[c])


_sc_call = pl.kernel(
    _sc_seg_sum,
    out_type=jax.ShapeDtypeStruct((NC, G, D), jnp.float32),
    mesh=plsc.VectorSubcoreMesh(core_axis_name="c", subcore_axis_name="s",
                                num_cores=NC, num_subcores=NS),
    scratch_types=[
        pltpu.VMEM((SLICE,), jnp.int32),
        pltpu.VMEM((SLICE,), jnp.int32),
        pltpu.VMEM((SLICES_PER_W * SLICE, D), jnp.float32),
        pltpu.VMEM((8, D), jnp.float32),
        pltpu.VMEM_SHARED((G, D), jnp.float32),
        pltpu.SemaphoreType.DMA,
        pltpu.SemaphoreType.DMA,
    ],
)


def _tc_partial(xh_ref, xtail_ref, batch_ref, part_ref, cnt_ref):
    batch = batch_ref[0, :]
    seg_ids = jax.lax.broadcasted_iota(jnp.int32, (G, N), 0)
    onehot = (batch[None, :] == seg_ids).astype(jnp.float32)
    cnt_ref[...] = jnp.sum(onehot, axis=1, keepdims=True) * jnp.ones(
        (1, D), jnp.float32)
    part = jnp.dot(onehot[:, :HEAD], xh_ref[...],
                   preferred_element_type=jnp.float32)
    part_ref[...] = part + jnp.dot(onehot[:, N_MAIN:], xtail_ref[...],
                                   preferred_element_type=jnp.float32)


def _tc_finish(sc_ref, tcp_ref, cnt_ref, u_ref, w1_ref, b1_ref,
               w2_ref, b2_ref, w3_ref, b3_ref, out_ref):
    seg_sum = jnp.sum(sc_ref[...], axis=0) + tcp_ref[...]
    seg_mean = seg_sum / jnp.maximum(cnt_ref[...], 1.0)
    cat = jnp.concatenate([u_ref[...], seg_mean], axis=1)
    h = jnp.dot(cat, w1_ref[...], preferred_element_type=jnp.float32)
    h = jnp.maximum(h + b1_ref[...], 0.0)
    h = jnp.dot(h, w2_ref[...], preferred_element_type=jnp.float32)
    h = jnp.maximum(h + b2_ref[...], 0.0)
    h = jnp.dot(h, w3_ref[...], preferred_element_type=jnp.float32)
    out_ref[...] = h + b3_ref[...]


def kernel(x, edge_index, u, batch, W1, b1, W2, b2, W3, b3):
    del edge_index
    sc_part = _sc_call(x, batch)
    tc_part, cnt = pl.pallas_call(
        _tc_partial,
        grid=(1,),
        in_specs=[
            pl.BlockSpec((HEAD, D), lambda i: (0, 0)),
            pl.BlockSpec((N_TAIL, D), lambda i: (0, 0)),
            pl.BlockSpec((1, N), lambda i: (0, 0)),
        ],
        out_specs=[
            pl.BlockSpec((G, D), lambda i: (0, 0)),
            pl.BlockSpec((G, D), lambda i: (0, 0)),
        ],
        out_shape=[
            jax.ShapeDtypeStruct((G, D), jnp.float32),
            jax.ShapeDtypeStruct((G, D), jnp.float32),
        ],
    )(x, x[N_MAIN:], batch.reshape(1, N))
    args = (sc_part, tc_part, cnt, u, W1.T, b1.reshape(1, H), W2.T,
            b2.reshape(1, H), W3.T, b3.reshape(1, OUT))
    return pl.pallas_call(
        _tc_finish,
        out_shape=jax.ShapeDtypeStruct((G, OUT), jnp.float32),
    )(*args)
